# initial kernel scaffold (unmeasured)
import jax
import jax.numpy as jnp
from jax import lax
from jax.experimental import pallas as pl
from jax.experimental.pallas import tpu as pltpu

N_DEV = 4
N_LAYERS = 3
N_HOPS = N_DEV - 1


def kernel(x, Win0, Wout0, Win1, Wout1, Win2, Wout2):
    b, d = x.shape
    rows_out = b // N_DEV

    def body(x_ref, win0_ref, wout0_ref, win1_ref, wout1_ref, win2_ref,
             wout2_ref, out_ref, comm_ref, send_sems, recv_sems):
        my_pos = lax.axis_index("i")
        left = lax.rem(my_pos + (N_DEV - 1), N_DEV)
        right = lax.rem(my_pos + 1, N_DEV)

        barrier_sem = pltpu.get_barrier_semaphore()
        for nbr in (left, right):
            pl.semaphore_signal(
                barrier_sem, inc=1,
                device_id=(nbr,), device_id_type=pl.DeviceIdType.MESH,
            )
        pl.semaphore_wait(barrier_sem, 2)

        wins = (win0_ref, win1_ref, win2_ref)
        wouts = (wout0_ref, wout1_ref, wout2_ref)

        x_val = x_ref[:, :]
        for l in range(N_LAYERS):
            h = jnp.maximum(
                jnp.dot(x_val, wins[l][:, :], preferred_element_type=jnp.float32),
                0.0,
            )
            partial = jnp.dot(h, wouts[l][:, :], preferred_element_type=jnp.float32)

            base = l * (N_HOPS + 1)
            comm_ref[base] = partial
            acc = partial
            for hop in range(N_HOPS):
                sem = l * N_HOPS + hop
                rdma = pltpu.make_async_remote_copy(
                    src_ref=comm_ref.at[base + hop],
                    dst_ref=comm_ref.at[base + hop + 1],
                    send_sem=send_sems.at[sem],
                    recv_sem=recv_sems.at[sem],
                    device_id=(right,),
                    device_id_type=pl.DeviceIdType.MESH,
                )
                rdma.start()
                rdma.wait()
                acc = acc + comm_ref[base + hop + 1]
            x_val = acc

        out_ref[:, :] = lax.dynamic_slice_in_dim(x_val, my_pos * rows_out, rows_out)

    return pl.pallas_call(
        body,
        out_shape=jax.ShapeDtypeStruct((rows_out, d), jnp.float32),
        in_specs=[pl.BlockSpec(memory_space=pltpu.VMEM)] * 7,
        out_specs=pl.BlockSpec(memory_space=pltpu.VMEM),
        scratch_shapes=[
            pltpu.VMEM((N_LAYERS * (N_HOPS + 1), b, d), jnp.float32),
            pltpu.SemaphoreType.DMA((N_LAYERS * N_HOPS,)),
            pltpu.SemaphoreType.DMA((N_LAYERS * N_HOPS,)),
        ],
        compiler_params=pltpu.CompilerParams(collective_id=0),
    )(x, Win0, Wout0, Win1, Wout1, Win2, Wout2)


# baseline (device time: 53058 ns/iter reference)
import jax
import jax.numpy as jnp
from jax import lax
from jax.experimental import pallas as pl
from jax.experimental.pallas import tpu as pltpu

N_DEV = 4
N_LAYERS = 3
N_HOPS = N_DEV - 1


def kernel(x, Win0, Wout0, Win1, Wout1, Win2, Wout2):
    b, d = x.shape
    rows_out = b // N_DEV

    def body(x_ref, win0_ref, wout0_ref, win1_ref, wout1_ref, win2_ref,
             wout2_ref, out_ref, comm_ref, send_sems, recv_sems):
        my_pos = lax.axis_index("i")
        left = lax.rem(my_pos + (N_DEV - 1), N_DEV)
        right = lax.rem(my_pos + 1, N_DEV)

        barrier_sem = pltpu.get_barrier_semaphore()
        for nbr in (left, right):
            pl.semaphore_signal(
                barrier_sem, inc=1,
                device_id=(nbr,), device_id_type=pl.DeviceIdType.MESH,
            )
        pl.semaphore_wait(barrier_sem, 2)

        wins = (win0_ref, win1_ref, win2_ref)
        wouts = (wout0_ref, wout1_ref, wout2_ref)

        x_val = x_ref[:, :]
        for l in range(N_LAYERS):
            h = jnp.maximum(
                jnp.dot(x_val, wins[l][:, :], preferred_element_type=jnp.float32),
                0.0,
            )
            partial = jnp.dot(h, wouts[l][:, :], preferred_element_type=jnp.float32)

            base = l * (N_HOPS + 1)
            comm_ref[base] = partial
            acc = partial
            for hop in range(N_HOPS):
                sem = l * N_HOPS + hop
                rdma = pltpu.make_async_remote_copy(
                    src_ref=comm_ref.at[base + hop],
                    dst_ref=comm_ref.at[base + hop + 1],
                    send_sem=send_sems.at[sem],
                    recv_sem=recv_sems.at[sem],
                    device_id=(right,),
                    device_id_type=pl.DeviceIdType.MESH,
                )
                rdma.start()
                rdma.wait()
                acc = acc + comm_ref[base + hop + 1]
            x_val = acc

        comm_ref[0] = x_val
        out_ref[:, :] = comm_ref[0, pl.ds(my_pos * rows_out, rows_out), :]

    return pl.pallas_call(
        body,
        out_shape=jax.ShapeDtypeStruct((rows_out, d), jnp.float32),
        in_specs=[pl.BlockSpec(memory_space=pltpu.VMEM)] * 7,
        out_specs=pl.BlockSpec(memory_space=pltpu.VMEM),
        scratch_shapes=[
            pltpu.VMEM((N_LAYERS * (N_HOPS + 1), b, d), jnp.float32),
            pltpu.SemaphoreType.DMA((N_LAYERS * N_HOPS,)),
            pltpu.SemaphoreType.DMA((N_LAYERS * N_HOPS,)),
        ],
        compiler_params=pltpu.CompilerParams(collective_id=0),
    )(x, Win0, Wout0, Win1, Wout1, Win2, Wout2)


# device time: 28272 ns/iter; 1.8767x vs baseline; 1.8767x over previous
import jax
import jax.numpy as jnp
from jax import lax
from jax.experimental import pallas as pl
from jax.experimental.pallas import tpu as pltpu

N_DEV = 4
HALF = 128


def kernel(x, Win0, Wout0, Win1, Wout1, Win2, Wout2):
    b, d = x.shape
    rows_out = b // N_DEV

    def body(x_ref, win0_ref, wout0_ref, win1_ref, wout1_ref, win2_ref,
             wout2_ref, out_ref, comm_ref, p_ref, rs_ref, send_sems, recv_sems):
        my_pos = lax.axis_index("i")
        partner_a = jnp.bitwise_xor(my_pos, 1)
        partner_b = jnp.bitwise_xor(my_pos, 3)

        barrier_sem = pltpu.get_barrier_semaphore()
        for nbr in (partner_a, partner_b):
            pl.semaphore_signal(
                barrier_sem, inc=1,
                device_id=(nbr,), device_id_type=pl.DeviceIdType.MESH,
            )
        pl.semaphore_wait(barrier_sem, 2)

        def exchange(send_slot, recv_slot, sem, partner, value):
            comm_ref[send_slot] = value
            rdma = pltpu.make_async_remote_copy(
                src_ref=comm_ref.at[send_slot],
                dst_ref=comm_ref.at[recv_slot],
                send_sem=send_sems.at[sem],
                recv_sem=recv_sems.at[sem],
                device_id=(partner,),
                device_id_type=pl.DeviceIdType.MESH,
            )
            rdma.start()
            return rdma

        wins = (win0_ref, win1_ref)
        wouts = (wout0_ref, wout1_ref)

        x_val = x_ref[:, :]
        for l in range(2):
            h = jnp.maximum(
                jnp.dot(x_val, wins[l][:, :], preferred_element_type=jnp.float32),
                0.0,
            )
            pa = jnp.dot(h, wouts[l][:, :HALF], preferred_element_type=jnp.float32)
            pb = jnp.dot(h, wouts[l][:, HALF:], preferred_element_type=jnp.float32)

            s = l * 8
            ra = exchange(s + 0, s + 1, l * 4 + 0, partner_a, pa)
            rb = exchange(s + 2, s + 3, l * 4 + 1, partner_b, pb)
            ra.wait()
            rb.wait()
            acc_a = pa + comm_ref[s + 1]
            acc_b = pb + comm_ref[s + 3]
            ra = exchange(s + 4, s + 5, l * 4 + 2, partner_b, acc_a)
            rb = exchange(s + 6, s + 7, l * 4 + 3, partner_a, acc_b)
            ra.wait()
            rb.wait()
            x_val = jnp.concatenate(
                [acc_a + comm_ref[s + 5], acc_b + comm_ref[s + 7]], axis=1
            )

        h = jnp.maximum(
            jnp.dot(x_val, win2_ref[:, :], preferred_element_type=jnp.float32),
            0.0,
        )
        p_ref[:, :] = jnp.dot(h, wout2_ref[:, :], preferred_element_type=jnp.float32)

        rdmas = []
        for i in range(1, N_DEV):
            t = lax.rem(my_pos + i, N_DEV)
            rdma = pltpu.make_async_remote_copy(
                src_ref=p_ref.at[pl.ds(t * rows_out, rows_out), :],
                dst_ref=rs_ref.at[3 - i],
                send_sem=send_sems.at[8 + (i - 1)],
                recv_sem=recv_sems.at[8 + (3 - i)],
                device_id=(t,),
                device_id_type=pl.DeviceIdType.MESH,
            )
            rdma.start()
            rdmas.append(rdma)
        for rdma in rdmas:
            rdma.wait_send()
            rdma.wait_recv()

        out_ref[:, :] = (
            p_ref[pl.ds(my_pos * rows_out, rows_out), :]
            + rs_ref[0] + rs_ref[1] + rs_ref[2]
        )

    return pl.pallas_call(
        body,
        out_shape=jax.ShapeDtypeStruct((rows_out, d), jnp.float32),
        in_specs=[pl.BlockSpec(memory_space=pltpu.VMEM)] * 7,
        out_specs=pl.BlockSpec(memory_space=pltpu.VMEM),
        scratch_shapes=[
            pltpu.VMEM((16, b, HALF), jnp.float32),
            pltpu.VMEM((b, d), jnp.float32),
            pltpu.VMEM((3, rows_out, d), jnp.float32),
            pltpu.SemaphoreType.DMA((11,)),
            pltpu.SemaphoreType.DMA((11,)),
        ],
        compiler_params=pltpu.CompilerParams(collective_id=0),
    )(x, Win0, Wout0, Win1, Wout1, Win2, Wout2)


# device time: 24703 ns/iter; 2.1478x vs baseline; 1.1445x over previous
import jax
import jax.numpy as jnp
from jax import lax
from jax.experimental import pallas as pl
from jax.experimental.pallas import tpu as pltpu

N_DEV = 4
HALF = 128


def kernel(x, Win0, Wout0, Win1, Wout1, Win2, Wout2):
    b, d = x.shape
    rows_out = b // N_DEV

    def body(x_ref, win0_ref, wout0_ref, win1_ref, wout1_ref, win2_ref,
             wout2_ref, out_ref, comm_ref, p_ref, rs_ref, send_sems, recv_sems):
        my_pos = lax.axis_index("i")
        partner_a = jnp.bitwise_xor(my_pos, 1)
        partner_b = jnp.bitwise_xor(my_pos, 3)

        barrier_sem = pltpu.get_barrier_semaphore()
        for nbr in (partner_a, partner_b):
            pl.semaphore_signal(
                barrier_sem, inc=1,
                device_id=(nbr,), device_id_type=pl.DeviceIdType.MESH,
            )
        pl.semaphore_wait(barrier_sem, 2)

        def exchange(send_slot, recv_slot, sem, partner, value):
            comm_ref[send_slot] = value.astype(jnp.bfloat16)
            rdma = pltpu.make_async_remote_copy(
                src_ref=comm_ref.at[send_slot],
                dst_ref=comm_ref.at[recv_slot],
                send_sem=send_sems.at[sem],
                recv_sem=recv_sems.at[sem],
                device_id=(partner,),
                device_id_type=pl.DeviceIdType.MESH,
            )
            rdma.start()
            return rdma

        wins = (win0_ref, win1_ref)
        wouts = (wout0_ref, wout1_ref)

        xb = x_ref[:, :].astype(jnp.bfloat16)
        for l in range(2):
            h = jnp.maximum(
                jnp.dot(xb, wins[l][:, :].astype(jnp.bfloat16),
                        preferred_element_type=jnp.float32),
                0.0,
            ).astype(jnp.bfloat16)
            wout_b = wouts[l][:, :].astype(jnp.bfloat16)
            s = l * 8
            pa = jnp.dot(h, wout_b[:, :HALF], preferred_element_type=jnp.float32)
            ra = exchange(s + 0, s + 1, l * 4 + 0, partner_a, pa)
            pb = jnp.dot(h, wout_b[:, HALF:], preferred_element_type=jnp.float32)
            rb = exchange(s + 2, s + 3, l * 4 + 1, partner_b, pb)
            ra.wait()
            acc_a = pa + comm_ref[s + 1].astype(jnp.float32)
            ra2 = exchange(s + 4, s + 5, l * 4 + 2, partner_b, acc_a)
            rb.wait()
            acc_b = pb + comm_ref[s + 3].astype(jnp.float32)
            rb2 = exchange(s + 6, s + 7, l * 4 + 3, partner_a, acc_b)
            ra2.wait()
            rb2.wait()
            xb = jnp.concatenate(
                [acc_a + comm_ref[s + 5].astype(jnp.float32),
                 acc_b + comm_ref[s + 7].astype(jnp.float32)],
                axis=1,
            ).astype(jnp.bfloat16)

        h = jnp.maximum(
            jnp.dot(xb, win2_ref[:, :].astype(jnp.bfloat16),
                    preferred_element_type=jnp.float32),
            0.0,
        ).astype(jnp.bfloat16)
        p_ref[:, :] = jnp.dot(
            h, wout2_ref[:, :].astype(jnp.bfloat16),
            preferred_element_type=jnp.float32,
        ).astype(jnp.bfloat16)

        rdmas = []
        for i in range(1, N_DEV):
            t = lax.rem(my_pos + i, N_DEV)
            rdma = pltpu.make_async_remote_copy(
                src_ref=p_ref.at[pl.ds(t * rows_out, rows_out), :],
                dst_ref=rs_ref.at[3 - i],
                send_sem=send_sems.at[8 + (i - 1)],
                recv_sem=recv_sems.at[8 + (3 - i)],
                device_id=(t,),
                device_id_type=pl.DeviceIdType.MESH,
            )
            rdma.start()
            rdmas.append(rdma)
        for rdma in rdmas:
            rdma.wait_send()
            rdma.wait_recv()

        out_ref[:, :] = (
            p_ref[pl.ds(my_pos * rows_out, rows_out), :].astype(jnp.float32)
            + rs_ref[0].astype(jnp.float32)
            + rs_ref[1].astype(jnp.float32)
            + rs_ref[2].astype(jnp.float32)
        )

    return pl.pallas_call(
        body,
        out_shape=jax.ShapeDtypeStruct((rows_out, d), jnp.float32),
        in_specs=[pl.BlockSpec(memory_space=pltpu.VMEM)] * 7,
        out_specs=pl.BlockSpec(memory_space=pltpu.VMEM),
        scratch_shapes=[
            pltpu.VMEM((16, b, HALF), jnp.bfloat16),
            pltpu.VMEM((b, d), jnp.bfloat16),
            pltpu.VMEM((3, rows_out, d), jnp.bfloat16),
            pltpu.SemaphoreType.DMA((11,)),
            pltpu.SemaphoreType.DMA((11,)),
        ],
        compiler_params=pltpu.CompilerParams(collective_id=0),
    )(x, Win0, Wout0, Win1, Wout1, Win2, Wout2)
